# baseline (speedup 1.0000x reference)
import functools

import jax
import jax.numpy as jnp
from jax import lax
from jax.experimental import pallas as pl
from jax.experimental.pallas import tpu as pltpu
from jax.experimental.pallas import tpu_sc as plsc

B, T, F = 100, 100, 128
N = B * T
H = 128
G4 = 4 * H
OUT = 12 * F
E = 320000

NC, NS = 2, 16
NW = NC * NS
EPW = E // NW
CH = 125
NCH = EPW // CH
EROWS = E // CH
RING = 2
IDXR = 4
RPS = 640
RLAST = N - (NS - 1) * RPS
DEGW = 16


def _sc_mesh():
    return plsc.VectorSubcoreMesh(
        core_axis_name="c", subcore_axis_name="s", num_cores=NC,
        num_subcores=NS)


def _sliced_copy(src, dst, s):
    off = pl.multiple_of(s * RPS, 8)

    @pl.when(s < NS - 1)
    def _():
        pltpu.sync_copy(src.at[pl.ds(off, RPS)], dst.at[pl.ds(off, RPS)])

    @pl.when(s == NS - 1)
    def _():
        pltpu.sync_copy(src.at[pl.ds(off, RLAST)], dst.at[pl.ds(off, RLAST)])


def _deg_body(dst_hbm, ones_hbm, zeros_hbm, out_hbm, idx_v, ones_v, acc_sh):
    c = lax.axis_index("c")
    s = lax.axis_index("s")
    w = c * NS + s
    _sliced_copy(zeros_hbm, acc_sh, s)
    pltpu.sync_copy(ones_hbm, ones_v)
    row0 = pl.multiple_of(w * NCH, 8)
    pltpu.sync_copy(dst_hbm.at[pl.ds(row0, NCH)], idx_v)
    plsc.subcore_barrier()

    def chunk(i, carry):
        pltpu.sync_copy(ones_v, acc_sh.at[idx_v.at[i]], add=True)
        return carry

    lax.fori_loop(0, NCH, chunk, 0)
    plsc.subcore_barrier()
    _sliced_copy(acc_sh, out_hbm.at[c], s)


def _msg_body(table_hbm, src_hbm, dst_hbm, zeros_hbm, out_hbm,
              sidx_v, didx_v, rows0, rows1,
              semg0, semg1, sems0, sems1, semi0, semi1, semi2, semi3, acc_sh):
    c = lax.axis_index("c")
    s = lax.axis_index("s")
    w = c * NS + s
    rows = [rows0, rows1]
    semg = [semg0, semg1]
    sems = [sems0, sems1]
    semi = [semi0, semi1, semi2, semi3]
    _sliced_copy(zeros_hbm, acc_sh, s)
    row0 = w * NCH
    pltpu.sync_copy(src_hbm.at[pl.ds(row0, NCH)], sidx_v)
    plsc.subcore_barrier()

    for b in range(IDXR - 1):
        pltpu.async_copy(dst_hbm.at[row0 + b], didx_v.at[b], semi[b])
    pltpu.async_copy(table_hbm.at[sidx_v.at[0]], rows[0], semg[0])

    def ring_round(i, carry):
        for b4 in range(4):
            g = i * 4 + b4
            r = b4 % 2
            q = b4 % IDXR
            qp = (b4 + 3) % IDXR
            pltpu.make_async_copy(dst_hbm.at[row0], didx_v.at[q],
                                  semi[q]).wait()
            pltpu.make_async_copy(table_hbm.at[sidx_v.at[g]], rows[r],
                                  semg[r]).wait()

            @pl.when(g > 0)
            def _():
                pltpu.make_async_copy(rows[1 - r], acc_sh.at[didx_v.at[q]],
                                      sems[1 - r]).wait()

            @pl.when(g + 1 < NCH)
            def _():
                pltpu.async_copy(table_hbm.at[sidx_v.at[g + 1]], rows[1 - r],
                                 semg[1 - r])

            pltpu.async_copy(rows[r], acc_sh.at[didx_v.at[q]], sems[r],
                             add=True)

            @pl.when(g + 3 < NCH)
            def _():
                pltpu.async_copy(dst_hbm.at[row0 + g + 3], didx_v.at[qp],
                                 semi[qp])
        return carry

    lax.fori_loop(0, NCH // 4, ring_round, 0)
    pltpu.make_async_copy(rows[(NCH - 1) % 2], acc_sh.at[didx_v.at[0]],
                          sems[(NCH - 1) % 2]).wait()
    plsc.subcore_barrier()
    _sliced_copy(acc_sh, out_hbm.at[c], s)


def _sc_deg(dst2):
    ones = jnp.ones((CH, DEGW), jnp.float32)
    zeros = jnp.zeros((N, DEGW), jnp.float32)
    return pl.kernel(
        _deg_body,
        out_type=jax.ShapeDtypeStruct((NC, N, DEGW), jnp.float32),
        mesh=_sc_mesh(),
        scratch_types=[
            pltpu.VMEM((NCH, CH), jnp.int32),
            pltpu.VMEM((CH, DEGW), jnp.float32),
            pltpu.VMEM_SHARED((N, DEGW), jnp.float32),
        ],
        compiler_params=pltpu.CompilerParams(use_tc_tiling_on_sc=False),
    )(dst2, ones, zeros)


def _sc_msg(table, src2, dst2):
    zeros = jnp.zeros((N, H), jnp.float32)
    return pl.kernel(
        _msg_body,
        out_type=jax.ShapeDtypeStruct((NC, N, H), jnp.float32),
        mesh=_sc_mesh(),
        scratch_types=[
            pltpu.VMEM((NCH, CH), jnp.int32),
            pltpu.VMEM((IDXR, CH), jnp.int32),
            pltpu.VMEM((CH, H), jnp.float32),
            pltpu.VMEM((CH, H), jnp.float32),
        ] + [pltpu.SemaphoreType.DMA] * 8 + [
            pltpu.VMEM_SHARED((N, H), jnp.float32),
        ],
        compiler_params=pltpu.CompilerParams(use_tc_tiling_on_sc=False),
    )(table, src2, dst2, zeros)


def _dense1_body(deg_ref, x_ref, w_ref, hwp_ref, dinv_ref):
    deg = deg_ref[0, :, 0:1] + deg_ref[1, :, 0:1] + 1.0
    dinv = jax.lax.rsqrt(deg)
    hw = jnp.dot(x_ref[...], w_ref[...], preferred_element_type=jnp.float32)
    hwp_ref[...] = hw * dinv
    dinv_ref[...] = dinv


def _gcn_post_body(acc_ref, hwp_ref, dinv_ref, b_ref, gamma_ref, beta_ref,
                   w2_ref, out_ref):
    dinv = dinv_ref[...]
    h = (acc_ref[0] + acc_ref[1] + hwp_ref[...]) * dinv + b_ref[...]
    mean = jnp.mean(h, axis=0, keepdims=True)
    var = jnp.mean((h - mean) ** 2, axis=0, keepdims=True)
    h = (h - mean) * jax.lax.rsqrt(var + 1e-5) * gamma_ref[...] + beta_ref[...]
    h = jnp.maximum(h, 0.0)
    out_ref[...] = jnp.dot(h, w2_ref[...],
                           preferred_element_type=jnp.float32) * dinv


def _gcn_final_body(acc_ref, hwp_ref, dinv_ref, b_ref, gamma_ref, beta_ref,
                    out_ref):
    dinv = dinv_ref[...]
    h = (acc_ref[0] + acc_ref[1] + hwp_ref[...]) * dinv + b_ref[...]
    mean = jnp.mean(h, axis=0, keepdims=True)
    var = jnp.mean((h - mean) ** 2, axis=0, keepdims=True)
    h = (h - mean) * jax.lax.rsqrt(var + 1e-5) * gamma_ref[...] + beta_ref[...]
    out_ref[...] = jnp.maximum(h, 0.0)


def _lstm_body(x_ref, w0_ref, b0_ref, w1_ref, b1_ref, wfc_ref, bfc_ref,
               out_ref):
    zero = jnp.zeros((B, H), dtype=jnp.float32)

    def step(t, carry):
        h0, c0, h1, c1 = carry
        xt = x_ref[t]

        def cell(xin, h, c, w_ref, b_ref):
            xcat = jnp.concatenate([xin, h], axis=1)
            gates = jnp.dot(xcat, w_ref[...],
                            preferred_element_type=jnp.float32) + b_ref[...]
            i = jax.nn.sigmoid(gates[:, 0 * H:1 * H])
            f = jax.nn.sigmoid(gates[:, 1 * H:2 * H])
            g = jnp.tanh(gates[:, 2 * H:3 * H])
            o = jax.nn.sigmoid(gates[:, 3 * H:4 * H])
            c = f * c + i * g
            h = o * jnp.tanh(c)
            return h, c

        h0, c0 = cell(xt, h0, c0, w0_ref, b0_ref)
        h1, c1 = cell(h0, h1, c1, w1_ref, b1_ref)
        return h0, c0, h1, c1

    _, _, h1, _ = jax.lax.fori_loop(0, T, step, (zero, zero, zero, zero))
    out_ref[...] = jnp.dot(h1, wfc_ref[...],
                           preferred_element_type=jnp.float32) + bfc_ref[...]


def _tc_call(body, out_shape):
    return pl.pallas_call(body, out_shape=out_shape)


def kernel(x, edge_index, W_gcn1, b_gcn1, gamma1, beta1, W_gcn2, b_gcn2,
           gamma2, beta2, Wih0, Whh0, bih0, bhh0, Wih1, Whh1, bih1, bhh1,
           W_fc, b_fc):
    x2 = x.reshape(N, F)
    src2 = edge_index[0].reshape(EROWS, CH)
    dst2 = edge_index[1].reshape(EROWS, CH)

    deg_parts = _sc_deg(dst2)

    hwp1, dinv = _tc_call(
        _dense1_body,
        (jax.ShapeDtypeStruct((N, H), jnp.float32),
         jax.ShapeDtypeStruct((N, 1), jnp.float32)),
    )(deg_parts, x2, W_gcn1)

    acc1 = _sc_msg(hwp1, src2, dst2)

    hwp2 = _tc_call(
        _gcn_post_body, jax.ShapeDtypeStruct((N, H), jnp.float32),
    )(acc1, hwp1, dinv, b_gcn1.reshape(1, H), gamma1.reshape(1, H),
      beta1.reshape(1, H), W_gcn2)

    acc2 = _sc_msg(hwp2, src2, dst2)

    h2 = _tc_call(
        _gcn_final_body, jax.ShapeDtypeStruct((N, H), jnp.float32),
    )(acc2, hwp2, dinv, b_gcn2.reshape(1, H), gamma2.reshape(1, H),
      beta2.reshape(1, H))

    x_tm = jnp.swapaxes(h2.reshape(B, T, H), 0, 1)

    w0 = jnp.concatenate([Wih0.T, Whh0.T], axis=0)
    w1 = jnp.concatenate([Wih1.T, Whh1.T], axis=0)
    b0 = (bih0 + bhh0).reshape(1, G4)
    b1 = (bih1 + bhh1).reshape(1, G4)
    out = _tc_call(
        _lstm_body, jax.ShapeDtypeStruct((B, OUT), jnp.float32),
    )(x_tm, w0, b0, w1, b1, W_fc.T, b_fc.reshape(1, OUT))

    return out.reshape(B, 12, F)


# speedup vs baseline: 1.1250x; 1.1250x over previous
import functools

import jax
import jax.numpy as jnp
from jax import lax
from jax.experimental import pallas as pl
from jax.experimental.pallas import tpu as pltpu
from jax.experimental.pallas import tpu_sc as plsc

B, T, F = 100, 100, 128
N = B * T
H = 128
G4 = 4 * H
OUT = 12 * F
E = 320000

NC, NS = 2, 16
NW = NC * NS
EPW = E // NW
CH = 125
NCH = EPW // CH
EROWS = E // CH
RING = 2
IDXR = 4
RPS = 640
RLAST = N - (NS - 1) * RPS
DEGW = 16


def _sc_mesh():
    return plsc.VectorSubcoreMesh(
        core_axis_name="c", subcore_axis_name="s", num_cores=NC,
        num_subcores=NS)


def _sliced_copy(src, dst, s):
    off = pl.multiple_of(s * RPS, 8)

    @pl.when(s < NS - 1)
    def _():
        pltpu.sync_copy(src.at[pl.ds(off, RPS)], dst.at[pl.ds(off, RPS)])

    @pl.when(s == NS - 1)
    def _():
        pltpu.sync_copy(src.at[pl.ds(off, RLAST)], dst.at[pl.ds(off, RLAST)])


def _deg_body(dst_hbm, ones_hbm, zeros_hbm, out_hbm, idx_v, ones_v, acc_sh):
    c = lax.axis_index("c")
    s = lax.axis_index("s")
    w = c * NS + s
    _sliced_copy(zeros_hbm, acc_sh, s)
    pltpu.sync_copy(ones_hbm, ones_v)
    row0 = pl.multiple_of(w * NCH, 8)
    pltpu.sync_copy(dst_hbm.at[pl.ds(row0, NCH)], idx_v)
    plsc.subcore_barrier()

    def chunk(i, carry):
        pltpu.sync_copy(ones_v, acc_sh.at[idx_v.at[i]], add=True)
        return carry

    lax.fori_loop(0, NCH, chunk, 0)
    plsc.subcore_barrier()
    _sliced_copy(acc_sh, out_hbm.at[c], s)


def _msg_body(table_hbm, src_hbm, dst_hbm, zeros_hbm, out_hbm,
              sidx_v, didx_v, rows0, rows1,
              semg0, semg1, semi0, semi1, acc_sh):
    c = lax.axis_index("c")
    s = lax.axis_index("s")
    w = c * NS + s
    rows = [rows0, rows1]
    semg = [semg0, semg1]
    semi = [semi0, semi1]
    _sliced_copy(zeros_hbm, acc_sh, s)
    row0 = w * NCH
    pltpu.sync_copy(src_hbm.at[pl.ds(row0, NCH)], sidx_v)
    plsc.subcore_barrier()

    for b in range(RING):
        pltpu.async_copy(dst_hbm.at[row0 + b], didx_v.at[b], semi[b])
        pltpu.async_copy(table_hbm.at[sidx_v.at[b]], rows[b], semg[b])

    def ring_round(i, carry):
        for b in range(RING):
            g = i * RING + b
            pltpu.make_async_copy(dst_hbm.at[row0 + b], didx_v.at[b],
                                  semi[b]).wait()
            pltpu.make_async_copy(table_hbm.at[sidx_v.at[g]], rows[b],
                                  semg[b]).wait()
            pltpu.sync_copy(rows[b], acc_sh.at[didx_v.at[b]], add=True)

            @pl.when(g + RING < NCH)
            def _():
                pltpu.async_copy(dst_hbm.at[row0 + g + RING], didx_v.at[b],
                                 semi[b])
                pltpu.async_copy(table_hbm.at[sidx_v.at[g + RING]], rows[b],
                                 semg[b])
        return carry

    lax.fori_loop(0, NCH // RING, ring_round, 0)
    plsc.subcore_barrier()
    _sliced_copy(acc_sh, out_hbm.at[c], s)


def _sc_deg(dst2):
    ones = jnp.ones((CH, DEGW), jnp.float32)
    zeros = jnp.zeros((N, DEGW), jnp.float32)
    return pl.kernel(
        _deg_body,
        out_type=jax.ShapeDtypeStruct((NC, N, DEGW), jnp.float32),
        mesh=_sc_mesh(),
        scratch_types=[
            pltpu.VMEM((NCH, CH), jnp.int32),
            pltpu.VMEM((CH, DEGW), jnp.float32),
            pltpu.VMEM_SHARED((N, DEGW), jnp.float32),
        ],
        compiler_params=pltpu.CompilerParams(use_tc_tiling_on_sc=False),
    )(dst2, ones, zeros)


def _sc_msg(table, src2, dst2):
    zeros = jnp.zeros((N, H), jnp.float32)
    return pl.kernel(
        _msg_body,
        out_type=jax.ShapeDtypeStruct((NC, N, H), jnp.float32),
        mesh=_sc_mesh(),
        scratch_types=[
            pltpu.VMEM((NCH, CH), jnp.int32),
            pltpu.VMEM((RING, CH), jnp.int32),
            pltpu.VMEM((CH, H), jnp.float32),
            pltpu.VMEM((CH, H), jnp.float32),
        ] + [pltpu.SemaphoreType.DMA] * 4 + [
            pltpu.VMEM_SHARED((N, H), jnp.float32),
        ],
        compiler_params=pltpu.CompilerParams(use_tc_tiling_on_sc=False),
    )(table, src2, dst2, zeros)


def _mm_body(x_ref, w_ref, out_ref):
    out_ref[...] = jnp.dot(x_ref[...], w_ref[...],
                           preferred_element_type=jnp.float32)


def _scale1_body(deg_ref, hw_ref, hwp_ref, dinv_ref):
    deg = deg_ref[0, :, 0:1] + deg_ref[1, :, 0:1] + 1.0
    dinv = jax.lax.rsqrt(deg)
    hwp_ref[...] = hw_ref[...] * dinv
    dinv_ref[...] = dinv


def _gcn_post_body(acc_ref, hwp_ref, dinv_ref, b_ref, gamma_ref, beta_ref,
                   w2_ref, out_ref):
    dinv = dinv_ref[...]
    h = (acc_ref[0] + acc_ref[1] + hwp_ref[...]) * dinv + b_ref[...]
    mean = jnp.mean(h, axis=0, keepdims=True)
    var = jnp.mean((h - mean) ** 2, axis=0, keepdims=True)
    h = (h - mean) * jax.lax.rsqrt(var + 1e-5) * gamma_ref[...] + beta_ref[...]
    h = jnp.maximum(h, 0.0)
    out_ref[...] = jnp.dot(h, w2_ref[...],
                           preferred_element_type=jnp.float32) * dinv


def _gcn_final_body(acc_ref, hwp_ref, dinv_ref, b_ref, gamma_ref, beta_ref,
                    out_ref):
    dinv = dinv_ref[...]
    h = (acc_ref[0] + acc_ref[1] + hwp_ref[...]) * dinv + b_ref[...]
    mean = jnp.mean(h, axis=0, keepdims=True)
    var = jnp.mean((h - mean) ** 2, axis=0, keepdims=True)
    h = (h - mean) * jax.lax.rsqrt(var + 1e-5) * gamma_ref[...] + beta_ref[...]
    out_ref[...] = jnp.maximum(h, 0.0)


def _lstm_body(x_ref, w0_ref, b0_ref, w1_ref, b1_ref, wfc_ref, bfc_ref,
               out_ref):
    zero = jnp.zeros((B, H), dtype=jnp.float32)

    def step(t, carry):
        h0, c0, h1, c1 = carry
        xt = x_ref[t]

        def cell(xin, h, c, w_ref, b_ref):
            xcat = jnp.concatenate([xin, h], axis=1)
            gates = jnp.dot(xcat, w_ref[...],
                            preferred_element_type=jnp.float32) + b_ref[...]
            i = jax.nn.sigmoid(gates[:, 0 * H:1 * H])
            f = jax.nn.sigmoid(gates[:, 1 * H:2 * H])
            g = jnp.tanh(gates[:, 2 * H:3 * H])
            o = jax.nn.sigmoid(gates[:, 3 * H:4 * H])
            c = f * c + i * g
            h = o * jnp.tanh(c)
            return h, c

        h0, c0 = cell(xt, h0, c0, w0_ref, b0_ref)
        h1, c1 = cell(h0, h1, c1, w1_ref, b1_ref)
        return h0, c0, h1, c1

    _, _, h1, _ = jax.lax.fori_loop(0, T, step, (zero, zero, zero, zero))
    out_ref[...] = jnp.dot(h1, wfc_ref[...],
                           preferred_element_type=jnp.float32) + bfc_ref[...]


def _tc_call(body, out_shape):
    return pl.pallas_call(body, out_shape=out_shape)


def kernel(x, edge_index, W_gcn1, b_gcn1, gamma1, beta1, W_gcn2, b_gcn2,
           gamma2, beta2, Wih0, Whh0, bih0, bhh0, Wih1, Whh1, bih1, bhh1,
           W_fc, b_fc):
    x2 = x.reshape(N, F)
    src2 = edge_index[0].reshape(EROWS, CH)
    dst2 = edge_index[1].reshape(EROWS, CH)

    deg_parts = _sc_deg(dst2)

    hw1 = _tc_call(
        _mm_body, jax.ShapeDtypeStruct((N, H), jnp.float32))(x2, W_gcn1)
    hwp1, dinv = _tc_call(
        _scale1_body,
        (jax.ShapeDtypeStruct((N, H), jnp.float32),
         jax.ShapeDtypeStruct((N, 1), jnp.float32)),
    )(deg_parts, hw1)

    acc1 = _sc_msg(hwp1, src2, dst2)

    hwp2 = _tc_call(
        _gcn_post_body, jax.ShapeDtypeStruct((N, H), jnp.float32),
    )(acc1, hwp1, dinv, b_gcn1.reshape(1, H), gamma1.reshape(1, H),
      beta1.reshape(1, H), W_gcn2)

    acc2 = _sc_msg(hwp2, src2, dst2)

    h2 = _tc_call(
        _gcn_final_body, jax.ShapeDtypeStruct((N, H), jnp.float32),
    )(acc2, hwp2, dinv, b_gcn2.reshape(1, H), gamma2.reshape(1, H),
      beta2.reshape(1, H))

    x_tm = jnp.swapaxes(h2.reshape(B, T, H), 0, 1)

    w0 = jnp.concatenate([Wih0.T, Whh0.T], axis=0)
    w1 = jnp.concatenate([Wih1.T, Whh1.T], axis=0)
    b0 = (bih0 + bhh0).reshape(1, G4)
    b1 = (bih1 + bhh1).reshape(1, G4)
    out = _tc_call(
        _lstm_body, jax.ShapeDtypeStruct((B, OUT), jnp.float32),
    )(x_tm, w0, b0, w1, b1, W_fc.T, b_fc.reshape(1, OUT))

    return out.reshape(B, 12, F)
